# Initial kernel scaffold; baseline (speedup 1.0000x reference)
import jax
import jax.numpy as jnp
from jax.experimental import pallas as pl


def kernel(x, edge_index, W1, b1, W2, b2):
    raise NotImplementedError("write your pallas kernel here")


# speedup vs baseline: 10.8350x; 10.8350x over previous
import functools

import jax
import jax.numpy as jnp
from jax import lax
from jax.experimental import pallas as pl
from jax.experimental.pallas import tpu as pltpu
from jax.experimental.pallas import tpu_sc as plsc

NC = 2
NS = 16
NW = NC * NS
CHUNK = 128
DEG_W = 16


def _degree_pallas(dstp, n, n_pad, n_chunks):
    rpt = n_pad // NS
    mesh = plsc.VectorSubcoreMesh(
        core_axis_name="c", subcore_axis_name="s",
        num_cores=NC, num_subcores=NS)

    @functools.partial(
        pl.kernel,
        out_type=jax.ShapeDtypeStruct((NC * n_pad, DEG_W), jnp.float32),
        mesh=mesh,
        scratch_types=[
            pltpu.VMEM_SHARED((n_pad, DEG_W), jnp.float32),
            pltpu.VMEM((CHUNK, DEG_W), jnp.float32),
            pltpu.VMEM((CHUNK,), jnp.int32),
        ],
    )
    def k(dst_hbm, out_hbm, acc, vbuf, didx):
        cid = lax.axis_index("c")
        sid = lax.axis_index("s")
        w = cid * NS + sid

        def z(i, c):
            vbuf[i, :] = jnp.zeros((DEG_W,), jnp.float32)
            return c
        lax.fori_loop(0, CHUNK, z, 0)
        for t in range(rpt // CHUNK):
            pltpu.sync_copy(vbuf,
                            acc.at[pl.ds(sid * rpt + t * CHUNK, CHUNK)])

        def o(i, c):
            vbuf[i, :] = jnp.ones((DEG_W,), jnp.float32)
            return c
        lax.fori_loop(0, CHUNK, o, 0)
        plsc.subcore_barrier()

        def body(j, c):
            base = (w * n_chunks + j) * CHUNK
            pltpu.sync_copy(dst_hbm.at[pl.ds(base, CHUNK)], didx)
            pltpu.sync_copy(vbuf, acc.at[didx], add=True)
            return c
        lax.fori_loop(0, n_chunks, body, 0)
        plsc.subcore_barrier()

        out_base = cid * n_pad + sid * rpt
        pltpu.sync_copy(acc.at[pl.ds(sid * rpt, rpt)],
                        out_hbm.at[pl.ds(out_base, rpt)])

    return k(dstp)


def _scatter_pallas(hp, srcp, dstp, n, n_pad, n_chunks, d):
    rpt = n_pad // NS
    mesh = plsc.VectorSubcoreMesh(
        core_axis_name="c", subcore_axis_name="s",
        num_cores=NC, num_subcores=NS)

    @functools.partial(
        pl.kernel,
        out_type=jax.ShapeDtypeStruct((NC * n_pad, d), jnp.float32),
        mesh=mesh,
        scratch_types=[
            pltpu.VMEM_SHARED((n_pad, d), jnp.float32),
            pltpu.VMEM((CHUNK, d), jnp.float32),
            pltpu.VMEM((CHUNK,), jnp.int32),
            pltpu.VMEM((CHUNK,), jnp.int32),
            pltpu.SemaphoreType.DMA,
        ],
    )
    def k(hp_hbm, src_hbm, dst_hbm, out_hbm, acc, rows, sidx, didx, sem):
        cid = lax.axis_index("c")
        sid = lax.axis_index("s")
        w = cid * NS + sid

        def z(t, c):
            i = t // (d // 16)
            j = t % (d // 16)
            rows[i, pl.ds(j * 16, 16)] = jnp.zeros((16,), jnp.float32)
            return c
        lax.fori_loop(0, CHUNK * (d // 16), z, 0)
        for t in range(rpt // CHUNK):
            pltpu.sync_copy(rows,
                            acc.at[pl.ds(sid * rpt + t * CHUNK, CHUNK)])
        plsc.subcore_barrier()

        def body(j, c):
            base = (w * n_chunks + j) * CHUNK
            pltpu.sync_copy(src_hbm.at[pl.ds(base, CHUNK)], sidx)
            pltpu.sync_copy(dst_hbm.at[pl.ds(base, CHUNK)], didx)
            pltpu.async_copy(hp_hbm.at[sidx], rows, sem).wait()
            pltpu.sync_copy(rows, acc.at[didx], add=True)
            return c
        lax.fori_loop(0, n_chunks, body, 0)
        plsc.subcore_barrier()

        out_base = cid * n_pad + sid * rpt
        pltpu.sync_copy(acc.at[pl.ds(sid * rpt, rpt)],
                        out_hbm.at[pl.ds(out_base, rpt)])

    return k(hp, srcp, dstp)


def _mm_scale_body(x_ref, w_ref, da_ref, db_ref, hp_ref):
    dinv = lax.rsqrt(da_ref[:, 0:1] + db_ref[:, 0:1] + 1.0)
    h = jnp.dot(x_ref[...], w_ref[...], preferred_element_type=jnp.float32)
    hp_ref[...] = h * dinv


def _mid_body(p0_ref, p1_ref, hp_ref, da_ref, db_ref, b_ref, w2_ref,
              h1_ref, h2p_ref):
    dinv = lax.rsqrt(da_ref[:, 0:1] + db_ref[:, 0:1] + 1.0)
    h1 = jnp.tanh((p0_ref[...] + p1_ref[...] + hp_ref[...]) * dinv + b_ref[...])
    h1_ref[...] = h1
    h = jnp.dot(h1, w2_ref[...], preferred_element_type=jnp.float32)
    h2p_ref[...] = h * dinv


def _final_body(p0_ref, p1_ref, hp_ref, da_ref, db_ref, b_ref, h2_ref):
    dinv = lax.rsqrt(da_ref[:, 0:1] + db_ref[:, 0:1] + 1.0)
    h2_ref[...] = (p0_ref[...] + p1_ref[...] + hp_ref[...]) * dinv + b_ref[...]


def kernel(x, edge_index, W1, b1, W2, b2):
    n, d = x.shape
    e = edge_index.shape[1]
    n_chunks = -(-e // (NW * CHUNK))
    e_pad = NW * CHUNK * n_chunks
    n_pad = -(-(n + 1) // (NS * CHUNK)) * (NS * CHUNK)
    br = 1000
    grid = (n // br,)

    src = edge_index[0]
    dst = edge_index[1]
    pad = e_pad - e
    srcp = jnp.concatenate([src, jnp.zeros((pad,), edge_index.dtype)])
    dstp = jnp.concatenate([dst, jnp.full((pad,), n, edge_index.dtype)])

    deg_parts = _degree_pallas(dstp, n, n_pad, n_chunks)
    da = deg_parts[:n]
    db = deg_parts[n_pad:n_pad + n]

    row_spec = pl.BlockSpec((br, d), lambda i: (i, 0))
    deg_spec = pl.BlockSpec((br, DEG_W), lambda i: (i, 0))
    w_spec = pl.BlockSpec((d, d), lambda i: (0, 0))
    b_spec = pl.BlockSpec((1, d), lambda i: (0, 0))
    row_shape = jax.ShapeDtypeStruct((n, d), jnp.float32)

    h1p = pl.pallas_call(
        _mm_scale_body,
        grid=grid,
        in_specs=[row_spec, w_spec, deg_spec, deg_spec],
        out_specs=row_spec,
        out_shape=row_shape,
    )(x, W1, da, db)

    parts1 = _scatter_pallas(h1p, srcp, dstp, n, n_pad, n_chunks, d)

    h1, h2p = pl.pallas_call(
        _mid_body,
        grid=grid,
        in_specs=[row_spec, row_spec, row_spec, deg_spec, deg_spec,
                  b_spec, w_spec],
        out_specs=[row_spec, row_spec],
        out_shape=[row_shape, row_shape],
    )(parts1[:n], parts1[n_pad:n_pad + n], h1p, da, db, b1.reshape(1, d), W2)

    parts2 = _scatter_pallas(h2p, srcp, dstp, n, n_pad, n_chunks, d)

    h2 = pl.pallas_call(
        _final_body,
        grid=grid,
        in_specs=[row_spec, row_spec, row_spec, deg_spec, deg_spec, b_spec],
        out_specs=row_spec,
        out_shape=row_shape,
    )(parts2[:n], parts2[n_pad:n_pad + n], h2p, da, db, b2.reshape(1, d))

    return (h1, h2)
